# baseline (speedup 1.0000x reference)
import jax
import jax.numpy as jnp
from jax.experimental import pallas as pl
from jax.experimental.pallas import tpu as pltpu

B, C, N, T = 2, 16, 1000, 10
E, H = 16, 8
D = E * T
DE = D + 1
HP = 256


def _fused_kernel(x_ref, gt_ref, wbs_ref, wbc_ref, bconv_ref,
                  w1_ref, wal1_ref, war1_ref, b1_ref,
                  w2_ref, wal2_ref, war2_ref, b2_ref, out_ref):
    bf = jnp.bfloat16
    x = x_ref[0]
    x1 = jnp.dot(x, wbs_ref[...], preferred_element_type=jnp.float32) + bconv_ref[0:1, :]
    x2 = jnp.dot(x, wbc_ref[...], preferred_element_type=jnp.float32) + bconv_ref[1:2, :]
    x2 = jnp.where(x2 >= 0, x2, 0.01 * x2)
    h0 = x1 + x2
    mask = jnp.where(gt_ref[...] != 0.0, 1.0, 0.0).astype(bf)

    def layer(h32, w_ref_, wal_ref_, war_ref_, b_ref_):
        h_bf = h32.astype(bf)
        feat_all = jnp.dot(h_bf, w_ref_[...],
                           preferred_element_type=jnp.float32).astype(bf)
        el_all = jnp.dot(h32, wal_ref_[...], preferred_element_type=jnp.float32)
        er_all = jnp.dot(h32, war_ref_[...], preferred_element_type=jnp.float32)
        elT = el_all.T
        m_col = jnp.max(elT, axis=1, keepdims=True)
        e1T = jnp.exp(elT - m_col).astype(bf)
        e2T = jnp.exp(0.2 * (elT - m_col)).astype(bf)
        nelT = (-elT).astype(bf)
        m_row = jnp.max(el_all, axis=0, keepdims=True)
        u = er_all + m_row
        cst = jnp.maximum(u, 0.2 * u)
        f1 = jnp.exp(u - cst).astype(bf)
        f2 = jnp.exp(0.2 * u - cst).astype(bf)
        er_bf = er_all.astype(bf)
        acc = None
        for hh in range(H):
            fb = feat_all[:, HP * hh:HP * (hh + 1)]
            cond = er_bf[:, hh:hh + 1] >= nelT[hh:hh + 1, :]
            num = jnp.where(cond,
                            f1[:, hh:hh + 1] * e1T[hh:hh + 1, :],
                            f2[:, hh:hh + 1] * e2T[hh:hh + 1, :]) * mask
            rq = jnp.dot(num, fb, preferred_element_type=jnp.float32)
            z = jnp.maximum(rq[:, D:D + 1], 1e-30)
            rst = rq[:, :DE] * (1.0 / z) + b_ref_[hh:hh + 1, :]
            rst = jnp.where(rst > 0, rst, jnp.exp(rst) - 1.0)
            a = rst * (1.0 / H)
            acc = a if acc is None else acc + a
        return acc

    h1 = layer(h0, w1_ref, wal1_ref, war1_ref, b1_ref)
    h2 = layer(h1, w2_ref, wal2_ref, war2_ref, b2_ref)
    out_ref[0] = h0[:, :D] + h2[:, :D]


def kernel(x, g, w_start, b_start, w_cat, b_cat, W1, al1, ar1, bias1, W2, al2, ar2, bias2):
    f32 = jnp.float32
    X = x.transpose(0, 2, 1, 3).reshape(B, N, C * T)
    gt = g.T

    eye_t = jnp.eye(T, dtype=f32)

    def conv_block(w):
        blk = jnp.einsum('ec,tu->cteu', w, eye_t).reshape(C * T, E * T)
        return jnp.pad(blk, ((0, 0), (0, 1)))

    wbs, wbc = conv_block(w_start), conv_block(w_cat)
    one_hot_de = (jnp.arange(DE) == D).astype(f32)
    bconv = jnp.stack([
        jnp.pad(jnp.repeat(b_start, T), (0, 1)) + one_hot_de,
        jnp.pad(jnp.repeat(b_cat, T), (0, 1)),
    ])

    ones_w = jnp.zeros((DE, H, HP), f32).at[D, :, D].set(1.0)

    def head_blocks(W):
        Wp = jnp.pad(W.reshape(D, H, D), ((0, 1), (0, 0), (0, HP - D)))
        return (Wp + ones_w).reshape(DE, H * HP).astype(jnp.bfloat16)

    def fold_attn(W, a):
        wal = jnp.einsum('dhe,he->dh', W.reshape(D, H, D), a)
        return jnp.pad(wal, ((0, 1), (0, 0)))

    def bias_ext(bias):
        return jnp.pad(bias.reshape(H, D), ((0, 0), (0, 1)))

    w1b, w2b = head_blocks(W1), head_blocks(W2)
    wal1, war1 = fold_attn(W1, al1), fold_attn(W1, ar1)
    wal2, war2 = fold_attn(W2, al2), fold_attn(W2, ar2)
    b1e, b2e = bias_ext(bias1), bias_ext(bias2)

    const = lambda *shape: pl.BlockSpec(shape, lambda b: tuple(0 for _ in shape))
    out = pl.pallas_call(
        _fused_kernel,
        grid=(B,),
        in_specs=[
            pl.BlockSpec((1, N, D), lambda b: (b, 0, 0)),
            const(N, N),
            const(D, DE), const(D, DE), const(2, DE),
            const(DE, H * HP), const(DE, H), const(DE, H), const(H, DE),
            const(DE, H * HP), const(DE, H), const(DE, H), const(H, DE),
        ],
        out_specs=pl.BlockSpec((1, N, D), lambda b: (b, 0, 0)),
        out_shape=jax.ShapeDtypeStruct((B, N, D), f32),
        compiler_params=pltpu.CompilerParams(dimension_semantics=("parallel",)),
    )(X, gt, wbs, wbc, bconv, w1b, wal1, war1, b1e, w2b, wal2, war2, b2e)

    return out.reshape(B, N, E, T).transpose(0, 2, 1, 3)


# speedup vs baseline: 1.0548x; 1.0548x over previous
import jax
import jax.numpy as jnp
from jax.experimental import pallas as pl
from jax.experimental.pallas import tpu as pltpu

B, C, N, T = 2, 16, 1000, 10
E, H = 16, 8
D = E * T
DE = D + 1
HP = 256


def _fused_kernel(x_ref, gt_ref, wbs_ref, wbc_ref, bconv_ref,
                  w1_ref, wal1_ref, war1_ref, b1_ref,
                  w2_ref, wal2_ref, war2_ref, b2_ref, out_ref):
    bf = jnp.bfloat16
    x = x_ref[0]
    x1 = jnp.dot(x, wbs_ref[...], preferred_element_type=jnp.float32) + bconv_ref[0:1, :]
    x2 = jnp.dot(x, wbc_ref[...], preferred_element_type=jnp.float32) + bconv_ref[1:2, :]
    x2 = jnp.where(x2 >= 0, x2, 0.01 * x2)
    h0 = x1 + x2
    mask = jnp.where(gt_ref[...] != 0.0, 1.0, 0.0).astype(bf)

    def layer(h32, w_ref_, wal_ref_, war_ref_, b_ref_):
        h_bf = h32.astype(bf)
        el_all = jnp.dot(h32, wal_ref_[...], preferred_element_type=jnp.float32)
        er_all = jnp.dot(h32, war_ref_[...], preferred_element_type=jnp.float32)
        elT = el_all.T
        m_col = jnp.max(elT, axis=1, keepdims=True)
        e1T = jnp.exp(elT - m_col).astype(bf)
        e2T = jnp.exp(0.2 * (elT - m_col)).astype(bf)
        nelT = (-elT).astype(bf)
        m_row = jnp.max(el_all, axis=0, keepdims=True)
        u = er_all + m_row
        cst = jnp.maximum(u, 0.2 * u)
        f1 = jnp.exp(u - cst).astype(bf)
        f2 = jnp.exp(0.2 * u - cst).astype(bf)
        er_bf = er_all.astype(bf)
        acc = None
        for hh in range(H):
            fb = jnp.dot(h_bf, w_ref_[:, HP * hh:HP * (hh + 1)],
                         preferred_element_type=jnp.float32).astype(bf)
            cond = er_bf[:, hh:hh + 1] >= nelT[hh:hh + 1, :]
            num = jnp.where(cond,
                            f1[:, hh:hh + 1] * e1T[hh:hh + 1, :],
                            f2[:, hh:hh + 1] * e2T[hh:hh + 1, :]) * mask
            rq = jnp.dot(num, fb, preferred_element_type=jnp.float32)
            z = jnp.maximum(rq[:, D:D + 1], 1e-30)
            rst = rq[:, :DE] * (1.0 / z) + b_ref_[hh:hh + 1, :]
            rst = jnp.where(rst > 0, rst, jnp.exp(rst) - 1.0)
            a = rst * (1.0 / H)
            acc = a if acc is None else acc + a
        return acc

    h1 = layer(h0, w1_ref, wal1_ref, war1_ref, b1_ref)
    h2 = layer(h1, w2_ref, wal2_ref, war2_ref, b2_ref)
    out_ref[0] = h0[:, :D] + h2[:, :D]


def kernel(x, g, w_start, b_start, w_cat, b_cat, W1, al1, ar1, bias1, W2, al2, ar2, bias2):
    f32 = jnp.float32
    X = x.transpose(0, 2, 1, 3).reshape(B, N, C * T)
    gt = g.T

    eye_t = jnp.eye(T, dtype=f32)

    def conv_block(w):
        blk = jnp.einsum('ec,tu->cteu', w, eye_t).reshape(C * T, E * T)
        return jnp.pad(blk, ((0, 0), (0, 1)))

    wbs, wbc = conv_block(w_start), conv_block(w_cat)
    one_hot_de = (jnp.arange(DE) == D).astype(f32)
    bconv = jnp.stack([
        jnp.pad(jnp.repeat(b_start, T), (0, 1)) + one_hot_de,
        jnp.pad(jnp.repeat(b_cat, T), (0, 1)),
    ])

    ones_w = jnp.zeros((DE, H, HP), f32).at[D, :, D].set(1.0)

    def head_blocks(W):
        Wp = jnp.pad(W.reshape(D, H, D), ((0, 1), (0, 0), (0, HP - D)))
        return (Wp + ones_w).reshape(DE, H * HP).astype(jnp.bfloat16)

    def fold_attn(W, a):
        wal = jnp.einsum('dhe,he->dh', W.reshape(D, H, D), a)
        return jnp.pad(wal, ((0, 1), (0, 0)))

    def bias_ext(bias):
        return jnp.pad(bias.reshape(H, D), ((0, 0), (0, 1)))

    w1b, w2b = head_blocks(W1), head_blocks(W2)
    wal1, war1 = fold_attn(W1, al1), fold_attn(W1, ar1)
    wal2, war2 = fold_attn(W2, al2), fold_attn(W2, ar2)
    b1e, b2e = bias_ext(bias1), bias_ext(bias2)

    const = lambda *shape: pl.BlockSpec(shape, lambda b: tuple(0 for _ in shape))
    out = pl.pallas_call(
        _fused_kernel,
        grid=(B,),
        in_specs=[
            pl.BlockSpec((1, N, D), lambda b: (b, 0, 0)),
            const(N, N),
            const(D, DE), const(D, DE), const(2, DE),
            const(DE, H * HP), const(DE, H), const(DE, H), const(H, DE),
            const(DE, H * HP), const(DE, H), const(DE, H), const(H, DE),
        ],
        out_specs=pl.BlockSpec((1, N, D), lambda b: (b, 0, 0)),
        out_shape=jax.ShapeDtypeStruct((B, N, D), f32),
        compiler_params=pltpu.CompilerParams(dimension_semantics=("parallel",)),
    )(X, gt, wbs, wbc, bconv, w1b, wal1, war1, b1e, w2b, wal2, war2, b2e)

    return out.reshape(B, N, E, T).transpose(0, 2, 1, 3)


# speedup vs baseline: 1.0556x; 1.0007x over previous
import jax
import jax.numpy as jnp
from jax.experimental import pallas as pl
from jax.experimental.pallas import tpu as pltpu

B, C, N, T = 2, 16, 1000, 10
E, H = 16, 8
D = E * T
DE = D + 1
HP = 256


def _fused_kernel(x_ref, gt_ref, wbs_ref, wbc_ref, bconv_ref,
                  w1_ref, wal1_ref, war1_ref, b1_ref,
                  w2_ref, wal2_ref, war2_ref, b2_ref, out_ref):
    bf = jnp.bfloat16
    x = x_ref[0]
    x1 = jnp.dot(x, wbs_ref[...], preferred_element_type=jnp.float32) + bconv_ref[0:1, :]
    x2 = jnp.dot(x, wbc_ref[...], preferred_element_type=jnp.float32) + bconv_ref[1:2, :]
    x2 = jnp.where(x2 >= 0, x2, 0.01 * x2)
    h0 = x1 + x2
    mask = jnp.where(gt_ref[...] != 0.0, 1.0, 0.0).astype(bf)

    def layer(h32, w_ref_, wal_ref_, war_ref_, b_ref_):
        h_bf = h32.astype(bf)
        el_all = jnp.dot(h32, wal_ref_[...], preferred_element_type=jnp.float32)
        er_all = jnp.dot(h32, war_ref_[...], preferred_element_type=jnp.float32)
        elT = el_all.T
        m_col = jnp.max(elT, axis=1, keepdims=True)
        e1T = jnp.exp(elT - m_col).astype(bf)
        e2T = jnp.exp(0.2 * (elT - m_col)).astype(bf)
        nelT = (-elT).astype(bf)
        m_row = jnp.max(el_all, axis=0, keepdims=True)
        u = er_all + m_row
        cst = jnp.maximum(u, 0.2 * u)
        f1 = jnp.exp(u - cst).astype(bf)
        f2 = jnp.exp(0.2 * u - cst).astype(bf)
        er_bf = er_all.astype(bf)
        acc = None
        for hh in range(H):
            fb = jnp.dot(h_bf, w_ref_[:, HP * hh:HP * (hh + 1)],
                         preferred_element_type=jnp.float32).astype(bf)
            cond = er_bf[:, hh:hh + 1] >= nelT[hh:hh + 1, :]
            num = jnp.where(cond,
                            f1[:, hh:hh + 1] * e1T[hh:hh + 1, :],
                            f2[:, hh:hh + 1] * e2T[hh:hh + 1, :]) * mask
            rq = jnp.dot(num, fb, preferred_element_type=jnp.float32)
            z = jnp.maximum(rq[:, D:D + 1], 1e-30)
            rst = rq[:, :DE] * (1.0 / z) + b_ref_[hh:hh + 1, :]
            rst = jnp.where(rst > 0, rst, jnp.exp(rst) - 1.0)
            a = rst * (1.0 / H)
            acc = a if acc is None else acc + a
        return acc

    h1 = layer(h0, w1_ref, wal1_ref, war1_ref, b1_ref)
    h2 = layer(h1, w2_ref, wal2_ref, war2_ref, b2_ref)
    out_ref[0] = h0[:, :D] + h2[:, :D]


def kernel(x, g, w_start, b_start, w_cat, b_cat, W1, al1, ar1, bias1, W2, al2, ar2, bias2):
    f32 = jnp.float32
    X = x.transpose(0, 2, 1, 3).reshape(B, N, C * T)
    gt = g.T

    eye_t = jnp.eye(T, dtype=f32)

    def conv_block(w):
        blk = jnp.einsum('ec,tu->cteu', w, eye_t).reshape(C * T, E * T)
        return jnp.pad(blk, ((0, 0), (0, 1)))

    wbs, wbc = conv_block(w_start), conv_block(w_cat)
    one_hot_de = (jnp.arange(DE) == D).astype(f32)
    bconv = jnp.stack([
        jnp.pad(jnp.repeat(b_start, T), (0, 1)) + one_hot_de,
        jnp.pad(jnp.repeat(b_cat, T), (0, 1)),
    ])

    ones_w = jnp.zeros((DE, H, HP), f32).at[D, :, D].set(1.0)

    def head_blocks(W):
        Wp = jnp.pad(W.reshape(D, H, D), ((0, 1), (0, 0), (0, HP - D)))
        return (Wp + ones_w).reshape(DE, H * HP).astype(jnp.bfloat16)

    def fold_attn(W, a):
        wal = jnp.einsum('dhe,he->dh', W.reshape(D, H, D), a)
        return jnp.pad(wal, ((0, 1), (0, 0)))

    def bias_ext(bias):
        return jnp.pad(bias.reshape(H, D), ((0, 0), (0, 1)))

    w1b, w2b = head_blocks(W1), head_blocks(W2)
    wal1, war1 = fold_attn(W1, al1), fold_attn(W1, ar1)
    wal2, war2 = fold_attn(W2, al2), fold_attn(W2, ar2)
    b1e, b2e = bias_ext(bias1), bias_ext(bias2)

    const = lambda *shape: pl.BlockSpec(shape, lambda b: tuple(0 for _ in shape))
    out = pl.pallas_call(
        _fused_kernel,
        grid=(B,),
        in_specs=[
            pl.BlockSpec((1, N, D), lambda b: (b, 0, 0)),
            const(N, N),
            const(D, DE), const(D, DE), const(2, DE),
            const(DE, H * HP), const(DE, H), const(DE, H), const(H, DE),
            const(DE, H * HP), const(DE, H), const(DE, H), const(H, DE),
        ],
        out_specs=pl.BlockSpec((1, N, D), lambda b: (b, 0, 0)),
        out_shape=jax.ShapeDtypeStruct((B, N, D), f32),
        compiler_params=pltpu.CompilerParams(dimension_semantics=("arbitrary",)),
    )(X, gt, wbs, wbc, bconv, w1b, wal1, war1, b1e, w2b, wal2, war2, b2e)

    return out.reshape(B, N, E, T).transpose(0, 2, 1, 3)


# speedup vs baseline: 1.1002x; 1.0422x over previous
import jax
import jax.numpy as jnp
from jax.experimental import pallas as pl
from jax.experimental.pallas import tpu as pltpu

B, C, N, T = 2, 16, 1000, 10
E, H = 16, 8
D = E * T
DE = D + 1
HP = 256


def _fused_kernel(x_ref, gt_ref, wbs_ref, wbc_ref, bconv_ref,
                  w1_ref, wal1_ref, war1_ref,
                  w2_ref, wal2_ref, war2_ref, out_ref):
    bf = jnp.bfloat16
    f32 = jnp.float32
    x = x_ref[0]
    x1 = jnp.dot(x, wbs_ref[...], preferred_element_type=f32) + bconv_ref[0:1, :]
    x2 = jnp.dot(x, wbc_ref[...], preferred_element_type=f32) + bconv_ref[1:2, :]
    x2 = jnp.where(x2 >= 0, x2, 0.01 * x2)
    h0 = x1 + x2
    mask = jnp.where(gt_ref[...] != 0.0, 1.0, 0.0).astype(bf)

    def layer(h_bf, w_ref_, wal_ref_, war_ref_):
        el_all = jnp.dot(h_bf, wal_ref_[...], preferred_element_type=f32)
        er_all = jnp.dot(h_bf, war_ref_[...], preferred_element_type=f32)
        elT = el_all.T
        m_col = jnp.max(elT, axis=1, keepdims=True)
        e1T = jnp.exp(elT - m_col).astype(bf)
        e2T = jnp.exp(0.2 * (elT - m_col)).astype(bf)
        nelT = (-elT).astype(bf)
        m_row = jnp.max(el_all, axis=0, keepdims=True)
        u = er_all + m_row
        cst = jnp.maximum(u, 0.2 * u)
        f1 = jnp.exp(u - cst).astype(bf)
        f2 = jnp.exp(0.2 * u - cst).astype(bf)
        er_bf = er_all.astype(bf)
        acc = None
        for hh in range(H):
            fb = jnp.dot(h_bf, w_ref_[:, HP * hh:HP * (hh + 1)],
                         preferred_element_type=f32).astype(bf)
            cond = er_bf[:, hh:hh + 1] >= nelT[hh:hh + 1, :]
            num = jnp.where(cond,
                            f1[:, hh:hh + 1] * e1T[hh:hh + 1, :],
                            f2[:, hh:hh + 1] * e2T[hh:hh + 1, :]) * mask
            rq = jnp.dot(num, fb, preferred_element_type=f32)
            z = jnp.maximum(rq[:, D:D + 1], 1e-30)
            rst = (rq[:, :DE] * (1.0 / z)).astype(bf)
            rst = jnp.where(rst > 0, rst, jnp.exp(rst) - bf(1.0))
            a = rst * bf(1.0 / H)
            acc = a if acc is None else acc + a
        return acc

    h1 = layer(h0.astype(bf), w1_ref, wal1_ref, war1_ref)
    h2 = layer(h1, w2_ref, wal2_ref, war2_ref)
    out_ref[0] = h0[:, :D] + h2[:, :D].astype(f32)


def kernel(x, g, w_start, b_start, w_cat, b_cat, W1, al1, ar1, bias1, W2, al2, ar2, bias2):
    f32 = jnp.float32
    X = x.transpose(0, 2, 1, 3).reshape(B, N, C * T)
    gt = g.T

    eye_t = jnp.eye(T, dtype=f32)

    def conv_block(w):
        blk = jnp.einsum('ec,tu->cteu', w, eye_t).reshape(C * T, E * T)
        return jnp.pad(blk, ((0, 0), (0, 1)))

    wbs, wbc = conv_block(w_start), conv_block(w_cat)
    one_hot_de = (jnp.arange(DE) == D).astype(f32)
    bconv = jnp.stack([
        jnp.pad(jnp.repeat(b_start, T), (0, 1)) + one_hot_de,
        jnp.pad(jnp.repeat(b_cat, T), (0, 1)),
    ])

    ones_w = jnp.zeros((DE, H, HP), f32).at[D, :, D].set(1.0)

    def head_blocks(W, bias):
        Wp = jnp.pad(W.reshape(D, H, D), ((0, 1), (0, 0), (0, HP - D)))
        Wp = Wp.at[D, :, :D].add(bias.reshape(H, D))
        return (Wp + ones_w).reshape(DE, H * HP).astype(jnp.bfloat16)

    def fold_attn(W, a):
        wal = jnp.einsum('dhe,he->dh', W.reshape(D, H, D), a)
        return jnp.pad(wal, ((0, 1), (0, 0))).astype(jnp.bfloat16)

    w1b, w2b = head_blocks(W1, bias1), head_blocks(W2, bias2)
    wal1, war1 = fold_attn(W1, al1), fold_attn(W1, ar1)
    wal2, war2 = fold_attn(W2, al2), fold_attn(W2, ar2)

    const = lambda *shape: pl.BlockSpec(shape, lambda b: tuple(0 for _ in shape))
    out = pl.pallas_call(
        _fused_kernel,
        grid=(B,),
        in_specs=[
            pl.BlockSpec((1, N, D), lambda b: (b, 0, 0)),
            const(N, N),
            const(D, DE), const(D, DE), const(2, DE),
            const(DE, H * HP), const(DE, H), const(DE, H),
            const(DE, H * HP), const(DE, H), const(DE, H),
        ],
        out_specs=pl.BlockSpec((1, N, D), lambda b: (b, 0, 0)),
        out_shape=jax.ShapeDtypeStruct((B, N, D), f32),
        compiler_params=pltpu.CompilerParams(dimension_semantics=("arbitrary",)),
    )(X, gt, wbs, wbc, bconv, w1b, wal1, war1, w2b, wal2, war2)

    return out.reshape(B, N, E, T).transpose(0, 2, 1, 3)
